# baseline (speedup 1.0000x reference)
import functools

import jax
import jax.numpy as jnp
from jax import lax
from jax.experimental import pallas as pl
from jax.experimental.pallas import tpu as pltpu
from jax.experimental.pallas import tpu_sc as plsc

_NC = 2
_NS = 16
_L = 16
_NW = _NC * _NS

_LUT_SIZE = 257
_LUT_PAD = 264

_RPC = 8


def _interp_chunk(cols, lanes, tab_c, tab_b, xbuf, ybuf):
    t_hi = 255.9999847412109375
    for r in range(_RPC):
        @plsc.parallel_loop(0, cols // _L, unroll=8)
        def _(i):
            xv = xbuf[r, pl.ds(i * _L, _L)]
            t = jnp.minimum(jnp.maximum(xv * 16.0 + 128.0, 0.0), t_hi)
            fi = t.astype(jnp.int32)
            fi2 = (fi << 4) + lanes
            c = plsc.load_gather(tab_c, [fi2])
            d = plsc.load_gather(tab_b, [fi2])
            ybuf[r, pl.ds(i * _L, _L)] = c + d * t


def _sc_body(rows_per_w, n_chunks, cols, x_hbm, lut_hbm, out_hbm,
             lut_v, tab_c, tab_b, xb0, xb1, yb0, yb1,
             sem_i0, sem_i1, sem_o0, sem_o1):
    wid = lax.axis_index("s") * _NC + lax.axis_index("c")
    base = wid * rows_per_w

    pltpu.sync_copy(lut_hbm, lut_v)

    lanes = lax.iota(jnp.int32, _L)
    for j in range(256 // _L):
        idxv = lanes + (j * _L)
        a = plsc.load_gather(lut_v, [idxv])
        hi = plsc.load_gather(lut_v, [idxv + 1])
        b = hi - a
        c = a - b * idxv.astype(jnp.float32)
        addr = idxv << 4
        for l in range(_L):
            plsc.store_scatter(tab_c, [addr + l], c)
            plsc.store_scatter(tab_b, [addr + l], b)

    def start_in(c, xb, sem):
        pltpu.async_copy(x_hbm.at[pl.ds(base + c * _RPC, _RPC), :], xb, sem)

    def wait_in(xb, sem):
        pltpu.make_async_copy(x_hbm.at[pl.ds(base, _RPC), :], xb, sem).wait()

    def start_out(c, yb, sem):
        pltpu.async_copy(yb, out_hbm.at[pl.ds(base + c * _RPC, _RPC), :], sem)

    def wait_out(yb, sem):
        pltpu.make_async_copy(yb, out_hbm.at[pl.ds(base, _RPC), :], sem).wait()

    n_outer = n_chunks // 2
    start_in(0, xb0, sem_i0)

    def outer(i, carry):
        c0 = i * 2
        start_in(c0 + 1, xb1, sem_i1)
        wait_in(xb0, sem_i0)

        @pl.when(i >= 1)
        def _():
            wait_out(yb0, sem_o0)

        _interp_chunk(cols, lanes, tab_c, tab_b, xb0, yb0)
        start_out(c0, yb0, sem_o0)

        @pl.when(i < n_outer - 1)
        def _():
            start_in(c0 + 2, xb0, sem_i0)

        wait_in(xb1, sem_i1)

        @pl.when(i >= 1)
        def _():
            wait_out(yb1, sem_o1)

        _interp_chunk(cols, lanes, tab_c, tab_b, xb1, yb1)
        start_out(c0 + 1, yb1, sem_o1)
        return carry

    lax.fori_loop(0, n_outer, outer, 0)
    wait_out(yb0, sem_o0)
    wait_out(yb1, sem_o1)


def kernel(x, lut):
    cols = x.shape[-1]
    rows = x.size // cols
    assert rows % (_NW * _RPC * 2) == 0 and cols % _L == 0
    rows_per_w = rows // _NW
    n_chunks = rows_per_w // _RPC

    lut_p = jnp.pad(lut, (0, _LUT_PAD - _LUT_SIZE))
    x2 = x.reshape(rows, cols)

    mesh = plsc.VectorSubcoreMesh(
        core_axis_name="c", subcore_axis_name="s",
        num_cores=_NC, num_subcores=_NS)

    body = functools.partial(_sc_body, rows_per_w, n_chunks, cols)
    run = pl.kernel(
        body,
        out_type=jax.ShapeDtypeStruct((rows, cols), jnp.float32),
        mesh=mesh,
        compiler_params=pltpu.CompilerParams(needs_layout_passes=False),
        scratch_types=[
            pltpu.VMEM((_LUT_PAD,), jnp.float32),
            pltpu.VMEM((256 * _L,), jnp.float32),
            pltpu.VMEM((256 * _L,), jnp.float32),
            pltpu.VMEM((_RPC, cols), jnp.float32),
            pltpu.VMEM((_RPC, cols), jnp.float32),
            pltpu.VMEM((_RPC, cols), jnp.float32),
            pltpu.VMEM((_RPC, cols), jnp.float32),
            pltpu.SemaphoreType.DMA,
            pltpu.SemaphoreType.DMA,
            pltpu.SemaphoreType.DMA,
            pltpu.SemaphoreType.DMA,
        ],
    )
    y = run(x2, lut_p)
    return y.reshape(x.shape)


# speedup vs baseline: 1.1187x; 1.1187x over previous
import functools

import jax
import jax.numpy as jnp
from jax import lax
from jax.experimental import pallas as pl
from jax.experimental.pallas import tpu as pltpu
from jax.experimental.pallas import tpu_sc as plsc

_NC = 2
_NS = 16
_L = 16
_NW = _NC * _NS

_LUT_SIZE = 257
_LUT_PAD = 264

_RPC = 8


def _interp_chunk(cols, tab_c, tab_b, xbuf, ybuf):
    t_hi = 255.9999847412109375

    @plsc.parallel_loop(0, _RPC * cols // _L, unroll=8)
    def _(i):
        xv = xbuf[pl.ds(i * _L, _L)]
        t = jnp.minimum(jnp.maximum(xv * 16.0 + 128.0, 0.0), t_hi)
        fi = t.astype(jnp.int32)
        c = plsc.load_gather(tab_c, [fi])
        d = plsc.load_gather(tab_b, [fi])
        ybuf[pl.ds(i * _L, _L)] = c + d * t


def _sc_body(rows_per_w, n_chunks, cols, x_hbm, lut_hbm, out_hbm,
             lut_v, tab_c, tab_b, xb0, xb1, yb0, yb1,
             sem_i0, sem_i1, sem_o0, sem_o1):
    wid = lax.axis_index("s") * _NC + lax.axis_index("c")
    base = wid * rows_per_w

    pltpu.sync_copy(lut_hbm, lut_v)

    lanes = lax.iota(jnp.int32, _L)
    for j in range(256 // _L):
        idxv = lanes + (j * _L)
        a = plsc.load_gather(lut_v, [idxv])
        hi = plsc.load_gather(lut_v, [idxv + 1])
        b = hi - a
        tab_c[pl.ds(j * _L, _L)] = a - b * idxv.astype(jnp.float32)
        tab_b[pl.ds(j * _L, _L)] = b

    def start_in(c, xb, sem):
        for r in range(_RPC):
            pltpu.async_copy(x_hbm.at[base + c * _RPC + r, :],
                             xb.at[pl.ds(r * cols, cols)], sem)

    def wait_in(xb, sem):
        for r in range(_RPC):
            pltpu.make_async_copy(x_hbm.at[base, :],
                                  xb.at[pl.ds(r * cols, cols)], sem).wait()

    def start_out(c, yb, sem):
        for r in range(_RPC):
            pltpu.async_copy(yb.at[pl.ds(r * cols, cols)],
                             out_hbm.at[base + c * _RPC + r, :], sem)

    def wait_out(yb, sem):
        for r in range(_RPC):
            pltpu.make_async_copy(yb.at[pl.ds(r * cols, cols)],
                                  out_hbm.at[base, :], sem).wait()

    n_outer = n_chunks // 2
    start_in(0, xb0, sem_i0)

    def outer(i, carry):
        c0 = i * 2
        start_in(c0 + 1, xb1, sem_i1)
        wait_in(xb0, sem_i0)

        @pl.when(i >= 1)
        def _():
            wait_out(yb0, sem_o0)

        _interp_chunk(cols, tab_c, tab_b, xb0, yb0)
        start_out(c0, yb0, sem_o0)

        @pl.when(i < n_outer - 1)
        def _():
            start_in(c0 + 2, xb0, sem_i0)

        wait_in(xb1, sem_i1)

        @pl.when(i >= 1)
        def _():
            wait_out(yb1, sem_o1)

        _interp_chunk(cols, tab_c, tab_b, xb1, yb1)
        start_out(c0 + 1, yb1, sem_o1)
        return carry

    lax.fori_loop(0, n_outer, outer, 0)
    wait_out(yb0, sem_o0)
    wait_out(yb1, sem_o1)


def kernel(x, lut):
    cols = x.shape[-1]
    rows = x.size // cols
    assert rows % (_NW * _RPC * 2) == 0 and cols % _L == 0
    rows_per_w = rows // _NW
    n_chunks = rows_per_w // _RPC

    lut_p = jnp.pad(lut, (0, _LUT_PAD - _LUT_SIZE))
    x2 = x.reshape(rows, cols)

    mesh = plsc.VectorSubcoreMesh(
        core_axis_name="c", subcore_axis_name="s",
        num_cores=_NC, num_subcores=_NS)

    body = functools.partial(_sc_body, rows_per_w, n_chunks, cols)
    run = pl.kernel(
        body,
        out_type=jax.ShapeDtypeStruct((rows, cols), jnp.float32),
        mesh=mesh,
        compiler_params=pltpu.CompilerParams(needs_layout_passes=False),
        scratch_types=[
            pltpu.VMEM((_LUT_PAD,), jnp.float32),
            pltpu.VMEM((_LUT_PAD,), jnp.float32),
            pltpu.VMEM((_LUT_PAD,), jnp.float32),
            pltpu.VMEM((_RPC * cols,), jnp.float32),
            pltpu.VMEM((_RPC * cols,), jnp.float32),
            pltpu.VMEM((_RPC * cols,), jnp.float32),
            pltpu.VMEM((_RPC * cols,), jnp.float32),
            pltpu.SemaphoreType.DMA,
            pltpu.SemaphoreType.DMA,
            pltpu.SemaphoreType.DMA,
            pltpu.SemaphoreType.DMA,
        ],
    )
    y = run(x2, lut_p)
    return y.reshape(x.shape)
